# Initial kernel scaffold; baseline (speedup 1.0000x reference)
import jax
import jax.numpy as jnp
from jax.experimental import pallas as pl


def kernel(state, temporal_feat, species_emb, q_W, q_b, k_W, k_b, tproj_W, form_coefs, form_gates_raw, holling_alpha_raw, mlp_W1, mlp_b1, mlp_W2, mlp_b2, mlp_W3, mlp_b3, r):
    raise NotImplementedError("write your pallas kernel here")


# speedup vs baseline: 1.9341x; 1.9341x over previous
import functools
import math

import jax
import jax.numpy as jnp
from jax.experimental import pallas as pl


_N = 64
_D = 32
_H = 32
_TOPK = 8
_TB = 8


def _gelu(x):
    return 0.5 * x * (1.0 + jax.lax.erf(x * (1.0 / math.sqrt(2.0))))


def _token_kernel(state_ref, tf_ref, sp_ref, tproj_ref, qwp_ref, kwp_ref,
                  qws_ref, kws_ref, qb_ref, kb_ref, wxi_ref, wxj_ref,
                  spA_ref, spC_ref, w2_ref, b2_ref, w3_ref,
                  wc0_ref, wc1_ref, wc2_ref, wc3_ref, wc4_ref,
                  mbias_ref, alpha_ref, r_ref,
                  lr_ref, attn_ref):
    f32 = jnp.float32
    N = _N
    sp = sp_ref[...]
    tproj = tproj_ref[...]
    qwp = qwp_ref[...]
    kwp = kwp_ref[...]
    qws = qws_ref[...]
    kws = kws_ref[...]
    qb = qb_ref[...]
    kb = kb_ref[...]
    wxi = wxi_ref[...]
    wxj = wxj_ref[...]
    spA = spA_ref[...]
    spC = spC_ref[...]
    w2 = w2_ref[...]
    b2 = b2_ref[...]
    w3 = w3_ref[...]
    wc0 = wc0_ref[...]
    wc1 = wc1_ref[...]
    wc2 = wc2_ref[...]
    wc3 = wc3_ref[...]
    wc4 = wc4_ref[...]
    mbias = mbias_ref[...]
    alpha = alpha_ref[...]
    r_row = r_ref[...]

    ii = jax.lax.broadcasted_iota(jnp.int32, (N, N), 0)
    jj = jax.lax.broadcasted_iota(jnp.int32, (N, N), 1)
    eye = jnp.where(ii == jj, 1.0, 0.0).astype(f32)
    ones_nn = jnp.ones((N, N), f32)

    dimnums_t = (((1,), (1,)), ((), ()))

    for t in range(_TB):
        s_row = state_ref[t:t + 1, :]
        tf = tf_ref[t]

        xj_b = jnp.broadcast_to(s_row, (N, N))
        diag_s = eye * xj_b
        xi_b = jnp.dot(diag_s, ones_nn, preferred_element_type=f32)

        proj = jnp.dot(tf + sp, tproj, preferred_element_type=f32)
        st_q = jnp.dot(diag_s, jnp.broadcast_to(qws, (N, _D)),
                       preferred_element_type=f32)
        st_k = jnp.dot(diag_s, jnp.broadcast_to(kws, (N, _D)),
                       preferred_element_type=f32)
        q = st_q + jnp.dot(proj, qwp, preferred_element_type=f32) + qb
        k = st_k + jnp.dot(proj, kwp, preferred_element_type=f32) + kb
        scores = jax.lax.dot_general(q, k, dimnums_t,
                                     preferred_element_type=f32)
        scores = scores * (1.0 / math.sqrt(_D))

        A = jnp.dot(diag_s, jnp.broadcast_to(wxi, (N, _H)),
                    preferred_element_type=f32) + spA
        C = jnp.dot(diag_s, jnp.broadcast_to(wxj, (N, _H)),
                    preferred_element_type=f32) + spC
        h1 = _gelu(jnp.broadcast_to(A.reshape(N, 1, _H), (N, N, _H)) +
                   jnp.broadcast_to(C.reshape(1, N, _H), (N, N, _H)))
        h1f = h1.reshape(N * N, _H)
        h2 = _gelu(jnp.dot(h1f, w2, preferred_element_type=f32) + b2)
        f4 = jnp.sum(h2 * jnp.broadcast_to(w3, (N * N, _H)), axis=1,
                     keepdims=True).reshape(N, N)

        holl = xj_b / (1.0 + alpha * xj_b)
        msgs = (wc0 * xj_b + wc1 * xi_b * xj_b + wc2 * holl +
                wc3 * xi_b * holl + wc4 * f4 + mbias)

        s3a = jnp.broadcast_to(scores.reshape(N, 1, N), (N, N, N))
        s3b = jnp.broadcast_to(scores.reshape(N, N, 1), (N, N, N))
        jp3 = jax.lax.broadcasted_iota(jnp.int32, (N, N, N), 2)
        j3 = jax.lax.broadcasted_iota(jnp.int32, (N, N, N), 1)
        beats = (s3a > s3b) | ((s3a == s3b) & (jp3 < j3))
        rank = jnp.sum(jnp.where(beats, 1.0, 0.0), axis=2)
        keep = rank < float(_TOPK)

        rowmax = jnp.max(scores, axis=1, keepdims=True)
        e = jnp.where(keep, jnp.exp(scores - rowmax), 0.0)
        z = jnp.sum(e, axis=1, keepdims=True)
        attn = e / z

        agg = jnp.sum(attn * msgs, axis=1, keepdims=True)
        lr_ref[t:t + 1, :] = r_row + agg.reshape(1, N)
        attn_ref[t] = attn


def _rep(shape):
    nd = len(shape)
    return pl.BlockSpec(shape, lambda i, _nd=nd: (0,) * _nd)


def kernel(state, temporal_feat, species_emb, q_W, q_b, k_W, k_b, tproj_W,
           form_coefs, form_gates_raw, holling_alpha_raw,
           mlp_W1, mlp_b1, mlp_W2, mlp_b2, mlp_W3, mlp_b3, r):
    B, T, N = state.shape
    D = species_emb.shape[1]
    H = mlp_W2.shape[0]
    BT = B * T

    gates = jax.nn.sigmoid(form_gates_raw)
    wc = form_coefs * gates
    alpha = (jax.nn.softplus(holling_alpha_raw) + 0.01).reshape(1, N)
    spA = species_emb @ mlp_W1[2:2 + D] + mlp_b1
    spC = species_emb @ mlp_W1[2 + D:2 + 2 * D]
    wxi = mlp_W1[0].reshape(1, H)
    wxj = mlp_W1[1].reshape(1, H)
    mbias = wc[4] * mlp_b3[0]
    qws = q_W[0].reshape(1, D)
    kws = k_W[0].reshape(1, D)
    qwp = q_W[1:]
    kwp = k_W[1:]
    w3 = mlp_W3.reshape(1, H)

    state2 = state.reshape(BT, N)
    tf2 = temporal_feat.reshape(BT, N, D)

    grid = (BT // _TB,)
    out_shape = (
        jax.ShapeDtypeStruct((BT, N), jnp.float32),
        jax.ShapeDtypeStruct((BT, N, N), jnp.float32),
    )
    in_specs = [
        pl.BlockSpec((_TB, N), lambda i: (i, 0)),
        pl.BlockSpec((_TB, N, D), lambda i: (i, 0, 0)),
        _rep((N, D)),
        _rep((D, D)),
        _rep((D, D)),
        _rep((D, D)),
        _rep((1, D)),
        _rep((1, D)),
        _rep((1, D)),
        _rep((1, D)),
        _rep((1, H)),
        _rep((1, H)),
        _rep((N, H)),
        _rep((N, H)),
        _rep((H, H)),
        _rep((1, H)),
        _rep((1, H)),
        _rep((N, N)),
        _rep((N, N)),
        _rep((N, N)),
        _rep((N, N)),
        _rep((N, N)),
        _rep((N, N)),
        _rep((1, N)),
        _rep((1, N)),
    ]
    out_specs = (
        pl.BlockSpec((_TB, N), lambda i: (i, 0)),
        pl.BlockSpec((_TB, N, N), lambda i: (i, 0, 0)),
    )

    lr2, attn2 = pl.pallas_call(
        _token_kernel,
        grid=grid,
        in_specs=in_specs,
        out_specs=out_specs,
        out_shape=out_shape,
    )(state2, tf2, species_emb, tproj_W, qwp, kwp, qws, kws,
      q_b.reshape(1, D), k_b.reshape(1, D), wxi, wxj, spA, spC,
      mlp_W2, mlp_b2.reshape(1, H), w3,
      wc[0], wc[1], wc[2], wc[3], wc[4], mbias, alpha, r.reshape(1, N))

    return lr2.reshape(B, T, N), attn2.reshape(B, T, N, N)


# speedup vs baseline: 5.0037x; 2.5871x over previous
import math

import jax
import jax.numpy as jnp
from jax.experimental import pallas as pl


_N = 64
_D = 32
_H = 32
_K = 8
_TB = 8


def _gelu(x):
    return 0.5 * x * (1.0 + jax.lax.erf(x * (1.0 / math.sqrt(2.0))))


_C00 = (((0,), (0,)), ((), ()))
_C10 = (((1,), (0,)), ((), ()))
_C11 = (((1,), (1,)), ((), ()))


def _token_kernel(state_ref, tf_ref, sp_ref, tproj_ref, qw_ref, kw_ref,
                  qb_ref, kb_ref, wxi_ref, wxj_ref,
                  spA_ref, spC_ref, w2_ref, b2_ref, w3_ref,
                  wc0_ref, wc1_ref, wc2_ref, wc3_ref, wc4_ref,
                  mbias_ref, alpha_ref, r_ref,
                  lr_ref, attn_ref):
    f32 = jnp.float32
    N = _N
    K = _K
    sp = sp_ref[...]
    tproj = tproj_ref[...]
    qb = qb_ref[...]
    kb = kb_ref[...]
    wxi = wxi_ref[...]
    wxj = wxj_ref[...]
    spA = spA_ref[...]
    spC = spC_ref[...]
    w2 = w2_ref[...]
    b2 = b2_ref[...]
    w3 = w3_ref[...]
    wc0 = wc0_ref[...]
    wc1 = wc1_ref[...]
    wc2 = wc2_ref[...]
    wc3 = wc3_ref[...]
    wc4 = wc4_ref[...]
    mbias = mbias_ref[...]
    alpha = alpha_ref[...]
    r_row = r_ref[...]

    i0 = jax.lax.broadcasted_iota(jnp.int32, (N, N), 0)
    i1 = jax.lax.broadcasted_iota(jnp.int32, (N, N), 1)
    eyeN = jnp.where(i0 == i1, 1.0, 0.0).astype(f32)
    cumU = jnp.where(i0 <= i1, 1.0, 0.0).astype(f32)
    jjf = i1.astype(f32)
    i8 = jax.lax.broadcasted_iota(jnp.int32, (_TB, _TB), 0)
    j8 = jax.lax.broadcasted_iota(jnp.int32, (_TB, _TB), 1)
    eyeT = jnp.where(i8 == j8, 1.0, 0.0).astype(f32)
    slot_i = jax.lax.broadcasted_iota(jnp.int32, (N, K, N), 1)

    state_blk = state_ref[...]
    sT = jax.lax.dot_general(state_blk, eyeT, _C00,
                             preferred_element_type=f32)

    for t in range(_TB):
        s_row = state_ref[t:t + 1, :]
        s_col = sT[:, t:t + 1]
        tf = tf_ref[t]

        proj = jax.lax.dot_general(tf + sp, tproj, _C10,
                                   preferred_element_type=f32)
        feats = jnp.concatenate([s_col, proj], axis=1)
        q = jax.lax.dot_general(feats, qw_ref[...], _C10,
                                preferred_element_type=f32) + qb
        k = jax.lax.dot_general(feats, kw_ref[...], _C10,
                                preferred_element_type=f32) + kb
        scores = jax.lax.dot_general(q, k, _C11, preferred_element_type=f32)
        scores = scores / (_D ** 0.5)

        removed = jnp.zeros((N, N), jnp.bool_)
        for _ in range(K):
            masked = jnp.where(removed, -jnp.inf, scores)
            m = jnp.max(masked, axis=1, keepdims=True)
            sel = masked == m
            minidx = jnp.min(jnp.where(sel, jjf, float(N)), axis=1,
                             keepdims=True)
            removed = removed | (sel & (jjf == minidx))
        keep = removed
        keepf = jnp.where(keep, 1.0, 0.0).astype(f32)

        kcum = jax.lax.dot_general(keepf, cumU, _C11,
                                   preferred_element_type=f32)
        slotv = (kcum - 0.5).astype(jnp.int32).reshape(N, 1, N)
        keep3 = keepf.reshape(N, 1, N)
        S3 = jnp.where((jnp.broadcast_to(slotv, (N, K, N)) == slot_i) &
                       (jnp.broadcast_to(keep3, (N, K, N)) > 0.0),
                       1.0, 0.0).astype(f32)
        S2 = S3.reshape(N * K, N)

        A = s_col * wxi + spA
        C = s_col * wxj + spC
        C_sel = jax.lax.dot_general(S2, C, _C10,
                                    preferred_element_type=f32)
        A_sel = jnp.broadcast_to(A.reshape(N, 1, _H),
                                 (N, K, _H)).reshape(N * K, _H)
        h1 = _gelu(A_sel + C_sel)
        h2 = _gelu(jax.lax.dot_general(h1, w2, _C10,
                                       preferred_element_type=f32) + b2)
        f4s = jnp.sum(h2.reshape(N, K, _H) * w3.reshape(1, 1, _H),
                      axis=2)

        xj_b = jnp.broadcast_to(s_row, (N, N))
        xi_b = jnp.broadcast_to(s_col, (N, N))
        holl = xj_b / (1.0 + alpha * xj_b)
        analytic = (wc0 * xj_b + wc1 * xi_b * xj_b + wc2 * holl +
                    wc3 * xi_b * holl + mbias)

        rowmax = jnp.max(scores, axis=1, keepdims=True)
        e = jnp.where(keep, jnp.exp(scores - rowmax), 0.0)
        z = jnp.sum(e, axis=1, keepdims=True)
        attn = e / z

        def sel_gather(x):
            return jnp.sum(S3 * x.reshape(N, 1, N), axis=2)

        msgs_sel = sel_gather(analytic) + sel_gather(wc4) * f4s
        attn_sel = sel_gather(e) / z
        agg_col = jnp.sum(attn_sel * msgs_sel, axis=1, keepdims=True)
        lr_row = jax.lax.dot_general(agg_col, eyeN, _C00,
                                     preferred_element_type=f32)

        lr_ref[t:t + 1, :] = r_row + lr_row
        attn_ref[t] = attn


def _rep(shape):
    nd = len(shape)
    return pl.BlockSpec(shape, lambda i, _nd=nd: (0,) * _nd)


def kernel(state, temporal_feat, species_emb, q_W, q_b, k_W, k_b, tproj_W,
           form_coefs, form_gates_raw, holling_alpha_raw,
           mlp_W1, mlp_b1, mlp_W2, mlp_b2, mlp_W3, mlp_b3, r):
    B, T, N = state.shape
    D = species_emb.shape[1]
    H = mlp_W2.shape[0]
    BT = B * T

    gates = jax.nn.sigmoid(form_gates_raw)
    wc = form_coefs * gates
    alpha = (jax.nn.softplus(holling_alpha_raw) + 0.01).reshape(1, N)
    spA = species_emb @ mlp_W1[2:2 + D] + mlp_b1
    spC = species_emb @ mlp_W1[2 + D:2 + 2 * D]
    wxi = mlp_W1[0].reshape(1, H)
    wxj = mlp_W1[1].reshape(1, H)
    mbias = wc[4] * mlp_b3[0]
    w3 = mlp_W3.reshape(1, H)

    state2 = state.reshape(BT, N)
    tf2 = temporal_feat.reshape(BT, N, D)

    grid = (BT // _TB,)
    out_shape = (
        jax.ShapeDtypeStruct((BT, N), jnp.float32),
        jax.ShapeDtypeStruct((BT, N, N), jnp.float32),
    )
    in_specs = [
        pl.BlockSpec((_TB, N), lambda i: (i, 0)),
        pl.BlockSpec((_TB, N, D), lambda i: (i, 0, 0)),
        _rep((N, D)),
        _rep((D, D)),
        _rep((1 + D, D)),
        _rep((1 + D, D)),
        _rep((1, D)),
        _rep((1, D)),
        _rep((1, H)),
        _rep((1, H)),
        _rep((N, H)),
        _rep((N, H)),
        _rep((H, H)),
        _rep((1, H)),
        _rep((1, H)),
        _rep((N, N)),
        _rep((N, N)),
        _rep((N, N)),
        _rep((N, N)),
        _rep((N, N)),
        _rep((N, N)),
        _rep((1, N)),
        _rep((1, N)),
    ]
    out_specs = (
        pl.BlockSpec((_TB, N), lambda i: (i, 0)),
        pl.BlockSpec((_TB, N, N), lambda i: (i, 0, 0)),
    )

    lr2, attn2 = pl.pallas_call(
        _token_kernel,
        grid=grid,
        in_specs=in_specs,
        out_specs=out_specs,
        out_shape=out_shape,
    )(state2, tf2, species_emb, tproj_W, q_W, k_W,
      q_b.reshape(1, D), k_b.reshape(1, D), wxi, wxj, spA, spC,
      mlp_W2, mlp_b2.reshape(1, H), w3,
      wc[0], wc[1], wc[2], wc[3], wc[4], mbias, alpha, r.reshape(1, N))

    return lr2.reshape(B, T, N), attn2.reshape(B, T, N, N)


# speedup vs baseline: 18.6144x; 3.7201x over previous
import math

import jax
import jax.numpy as jnp
from jax.experimental import pallas as pl


_N = 64
_D = 32
_H = 32
_K = 8
_TB = 8
_M = _TB * _N


def _gelu(x):
    return 0.5 * x * (1.0 + jax.lax.erf(x * (1.0 / math.sqrt(2.0))))


_C10 = (((1,), (0,)), ((), ()))
_C11 = (((1,), (1,)), ((), ()))
_BMM = (((2,), (2,)), ((0,), (0,)))


def _token_kernel(state_ref, tf_ref, sp_ref, tproj_ref, qw_ref, kw_ref,
                  qb_ref, kb_ref, wxi_ref, wxj_ref,
                  spA_ref, spC_ref, w2_ref, b2_ref, w3_ref,
                  wc0_ref, wc1_ref, wc2_ref, wc3_ref, wc4_ref,
                  mbias_ref, alpha_ref, r_ref,
                  lr_ref, attn_ref):
    f32 = jnp.float32
    N = _N
    K = _K
    M = _M
    qb = qb_ref[...]
    kb = kb_ref[...]
    wxi = wxi_ref[...]
    wxj = wxj_ref[...]
    w2 = w2_ref[...]
    b2 = b2_ref[...]
    w3 = w3_ref[...]
    alpha = alpha_ref[...]
    r_row = r_ref[...]

    i0 = jax.lax.broadcasted_iota(jnp.int32, (N, N), 0)
    i1 = jax.lax.broadcasted_iota(jnp.int32, (N, N), 1)
    cumU = jnp.where(i1 <= i0, 1.0, 0.0).astype(f32)
    jjf = jax.lax.broadcasted_iota(jnp.int32, (M, N), 1).astype(f32)
    slot_i = jax.lax.broadcasted_iota(jnp.int32, (M, K, N), 1)

    def tile_tok(x):
        a, b = x.shape
        return jnp.broadcast_to(x.reshape(1, a, b), (_TB, a, b)).reshape(M, b)

    sp_t = tile_tok(sp_ref[...])
    spA_t = tile_tok(spA_ref[...])
    spC_t = tile_tok(spC_ref[...])
    wc0 = tile_tok(wc0_ref[...])
    wc1 = tile_tok(wc1_ref[...])
    wc2 = tile_tok(wc2_ref[...])
    wc3 = tile_tok(wc3_ref[...])
    wc4 = tile_tok(wc4_ref[...])
    mbias = tile_tok(mbias_ref[...])

    state_blk = state_ref[...]
    t0 = jax.lax.broadcasted_iota(jnp.int32, (_TB, _TB), 0)
    t1 = jax.lax.broadcasted_iota(jnp.int32, (_TB, _TB), 1)
    eyeT = jnp.where(t0 == t1, 1.0, 0.0).astype(f32)
    sT = jax.lax.dot_general(state_blk, eyeT, (((0,), (0,)), ((), ())),
                             preferred_element_type=f32)
    xi_b = jnp.concatenate(
        [jnp.broadcast_to(sT[:, t:t + 1], (N, N)) for t in range(_TB)],
        axis=0)
    s_col = xi_b[:, 0:1]
    xj_b = jnp.broadcast_to(state_blk.reshape(_TB, 1, N),
                            (_TB, N, N)).reshape(M, N)

    tf = tf_ref[...].reshape(M, _D)
    proj = jax.lax.dot_general(tf + sp_t, tproj_ref[...], _C10,
                               preferred_element_type=f32)
    feats = jnp.concatenate([s_col, proj], axis=1)
    q = jax.lax.dot_general(feats, qw_ref[...], _C10,
                            preferred_element_type=f32) + qb
    k = jax.lax.dot_general(feats, kw_ref[...], _C10,
                            preferred_element_type=f32) + kb
    scores = jax.lax.dot_general(q.reshape(_TB, N, _D), k.reshape(_TB, N, _D),
                                 _BMM, preferred_element_type=f32)
    scores = (scores / (_D ** 0.5)).reshape(M, N)

    removed = jnp.zeros((M, N), jnp.bool_)
    for _ in range(K):
        masked = jnp.where(removed, -jnp.inf, scores)
        m = jnp.max(masked, axis=1, keepdims=True)
        sel = masked == m
        minidx = jnp.min(jnp.where(sel, jjf, float(N)), axis=1, keepdims=True)
        removed = removed | (sel & (jjf == minidx))
    keep = removed
    keepf = jnp.where(keep, 1.0, 0.0).astype(f32)

    kcum = jax.lax.dot_general(keepf, cumU, _C11,
                               preferred_element_type=f32)
    slotv = (kcum - 0.5).astype(jnp.int32).reshape(M, 1, N)
    keep3 = keepf.reshape(M, 1, N)
    S3 = jnp.where((jnp.broadcast_to(slotv, (M, K, N)) == slot_i) &
                   (jnp.broadcast_to(keep3, (M, K, N)) > 0.0),
                   1.0, 0.0).astype(f32)

    A = s_col * wxi + spA_t
    C = s_col * wxj + spC_t
    C_sel = jax.lax.dot_general(S3.reshape(_TB, N * K, N),
                                C.reshape(_TB, N, _H),
                                (((2,), (1,)), ((0,), (0,))),
                                preferred_element_type=f32)
    A_sel = jnp.broadcast_to(A.reshape(M, 1, _H), (M, K, _H))
    h1 = _gelu(A_sel.reshape(M * K, _H) + C_sel.reshape(M * K, _H))
    h2 = _gelu(jax.lax.dot_general(h1, w2, _C10,
                                   preferred_element_type=f32) + b2)
    f4s = jnp.sum(h2.reshape(M, K, _H) * w3.reshape(1, 1, _H),
                  axis=2)

    f4d = jnp.sum(S3 * f4s.reshape(M, K, 1), axis=1)

    holl = xj_b / (1.0 + alpha * xj_b)
    msgs = (wc0 * xj_b + wc1 * xi_b * xj_b + wc2 * holl +
            wc3 * xi_b * holl + wc4 * f4d + mbias)

    rowmax = jnp.max(scores, axis=1, keepdims=True)
    e = jnp.where(keep, jnp.exp(scores - rowmax), 0.0)
    z = jnp.sum(e, axis=1, keepdims=True)
    attn = e / z

    agg = jnp.sum((attn * msgs).reshape(_TB, N, N), axis=2)
    lr_ref[...] = r_row + agg
    attn_ref[...] = attn.reshape(_TB, N, N)


def _rep(shape):
    nd = len(shape)
    return pl.BlockSpec(shape, lambda i, _nd=nd: (0,) * _nd)


def kernel(state, temporal_feat, species_emb, q_W, q_b, k_W, k_b, tproj_W,
           form_coefs, form_gates_raw, holling_alpha_raw,
           mlp_W1, mlp_b1, mlp_W2, mlp_b2, mlp_W3, mlp_b3, r):
    B, T, N = state.shape
    D = species_emb.shape[1]
    H = mlp_W2.shape[0]
    BT = B * T

    gates = jax.nn.sigmoid(form_gates_raw)
    wc = form_coefs * gates
    alpha = (jax.nn.softplus(holling_alpha_raw) + 0.01).reshape(1, N)
    spA = species_emb @ mlp_W1[2:2 + D] + mlp_b1
    spC = species_emb @ mlp_W1[2 + D:2 + 2 * D]
    wxi = mlp_W1[0].reshape(1, H)
    wxj = mlp_W1[1].reshape(1, H)
    mbias = wc[4] * mlp_b3[0]
    w3 = mlp_W3.reshape(1, H)

    state2 = state.reshape(BT, N)
    tf2 = temporal_feat.reshape(BT, N, D)

    grid = (BT // _TB,)
    out_shape = (
        jax.ShapeDtypeStruct((BT, N), jnp.float32),
        jax.ShapeDtypeStruct((BT, N, N), jnp.float32),
    )
    in_specs = [
        pl.BlockSpec((_TB, N), lambda i: (i, 0)),
        pl.BlockSpec((_TB, N, D), lambda i: (i, 0, 0)),
        _rep((N, D)),
        _rep((D, D)),
        _rep((1 + D, D)),
        _rep((1 + D, D)),
        _rep((1, D)),
        _rep((1, D)),
        _rep((1, H)),
        _rep((1, H)),
        _rep((N, H)),
        _rep((N, H)),
        _rep((H, H)),
        _rep((1, H)),
        _rep((1, H)),
        _rep((N, N)),
        _rep((N, N)),
        _rep((N, N)),
        _rep((N, N)),
        _rep((N, N)),
        _rep((N, N)),
        _rep((1, N)),
        _rep((1, N)),
    ]
    out_specs = (
        pl.BlockSpec((_TB, N), lambda i: (i, 0)),
        pl.BlockSpec((_TB, N, N), lambda i: (i, 0, 0)),
    )

    lr2, attn2 = pl.pallas_call(
        _token_kernel,
        grid=grid,
        in_specs=in_specs,
        out_specs=out_specs,
        out_shape=out_shape,
    )(state2, tf2, species_emb, tproj_W, q_W, k_W,
      q_b.reshape(1, D), k_b.reshape(1, D), wxi, wxj, spA, spC,
      mlp_W2, mlp_b2.reshape(1, H), w3,
      wc[0], wc[1], wc[2], wc[3], wc[4], mbias, alpha, r.reshape(1, N))

    return lr2.reshape(B, T, N), attn2.reshape(B, T, N, N)
